# baseline (speedup 1.0000x reference)
import jax
import jax.numpy as jnp
from jax import lax
from jax.experimental import pallas as pl
from jax.experimental.pallas import tpu as pltpu

VTH_ = 1.0
T_ = 8

F32 = jnp.float32

S0, O0, M0, R0 = 18, 24, 288, 336
S1, O1, M1, R1 = 10, 16, 80, 112
SI, MI = 66, 2112
RH = 4356
OH = 72
REO = 2184


def _rowmask(m, s, v, c):
    r = lax.broadcasted_iota(jnp.int32, (m, c), 0)
    return (r % s) < v


def _aff_clip(acc, af_ref):
    return jnp.clip(acc * af_ref[0:1, :] + af_ref[1:2, :], 0.0, VTH_)


def _conv1_phase(src, w_ref, a, b, s, o, m):
    acc = None
    for di in range(3):
        qa = a + di - 1
        pa, du = qa & 1, (qa - (qa & 1)) // 2
        for dj in range(3):
            qb = b + dj - 1
            pb, dv = qb & 1, (qb - (qb & 1)) // 2
            st = o + s * du + dv
            d = jnp.dot(src[pa * 2 + pb, st:st + m, :], w_ref[di * 3 + dj],
                        preferred_element_type=F32)
            acc = d if acc is None else acc + d
    return acc


def _conv2_full(src, w_ref, s, o, m):
    acc = None
    for di in range(3):
        q = di - 1
        pa, du = q & 1, (q - (q & 1)) // 2
        for dj in range(3):
            q2 = dj - 1
            pb, dv = q2 & 1, (q2 - (q2 & 1)) // 2
            st = o + s * du + dv
            d = jnp.dot(src[pa * 2 + pb, st:st + m, :], w_ref[di * 3 + dj],
                        preferred_element_type=F32)
            acc = d if acc is None else acc + d
    return acc


def _convt_phase(s2, w_ref, e, f, s, o, m):
    acc = None
    for di in range(3):
        if (di & 1) == (e & 1):
            continue
        du = (e + di - 1) // 2
        for dj in range(3):
            if (dj & 1) == (f & 1):
                continue
            dv = (f + dj - 1) // 2
            st = o + s * du + dv
            d = jnp.dot(s2[st:st + m, :], w_ref[di * 3 + dj],
                        preferred_element_type=F32)
            acc = d if acc is None else acc + d
    return acc


def _main_kernel(x9, w27, af_di1, wdi2, af_di2, wwx, af_wx, wwxn, af_wxn,
                 ws1, af_s1, ws2, af_s2, ws1n, af_s1n, ws2n, af_s2n,
                 wtr, wtrn, wds, af_ds, winc, af_inc, out,
                 E, Ob, H2, HN, A0, S1b, S2b, I0, A1, S1n, S2n, I1):
    m18 = _rowmask(M0, S0, 16, 64)
    m18s2 = _rowmask(M0, S0, 16, 256)
    m18s1 = _rowmask(M0, S0, 16, 128)
    m10 = _rowmask(M1, S1, 8, 64)
    m10s1 = _rowmask(M1, S1, 8, 128)
    m10s2 = _rowmask(M1, S1, 8, 256)
    m66 = _rowmask(MI, SI, 32, 64)
    zf = jnp.float32(0.0)

    for r4 in (A0, S1b, A1, S1n):
        r4[...] = jnp.zeros_like(r4)
    S2b[...] = jnp.zeros_like(S2b)
    S2n[...] = jnp.zeros_like(S2n)
    HN[...] = jnp.zeros_like(HN)
    H2[0:OH, :] = jnp.zeros((OH, 64), F32)
    H2[OH + MI:RH, :] = jnp.zeros((RH - OH - MI, 64), F32)
    E[2178:REO, :] = jnp.zeros((REO - 2178, 64), F32)
    Ob[2178:REO, :] = jnp.zeros((REO - 2178, 64), F32)

    p = lax.broadcasted_iota(jnp.int32, (RH, 64), 0)
    pi, pj = p // SI, p % SI
    mint = (pi >= 1) & (pi < 65) & (pj >= 1) & (pj < 65)
    h1 = jnp.dot(x9[0], w27[...], preferred_element_type=F32)
    h1 = jnp.where(mint, _aff_clip(h1, af_di1), zf)
    v3 = h1.reshape(2178, 2, 64)
    E[0:2178, :] = v3[:, 0, :]
    Ob[0:2178, :] = v3[:, 1, :]

    acc = None
    for di in range(3):
        for dj in range(3):
            off = di * SI + dj
            src = Ob if (off & 1) else E
            b0 = off // 2
            d = jnp.dot(src[b0:b0 + MI, :], wdi2[di * 3 + dj],
                        preferred_element_type=F32)
            acc = d if acc is None else acc + d
    m66v = _rowmask(MI, SI, 32, 64)
    hv = jnp.where(m66v, _aff_clip(acc, af_di2), zf)
    H2[OH:OH + MI, :] = hv

    h4 = hv.reshape(32, 33, 2, 64)
    hm = (h4[:, :, 0, :] + h4[:, :, 1, :]) * 0.5
    hm2 = hm.reshape(16, 2, 33, 64)
    hm3 = (hm2[:, 0, :, :] + hm2[:, 1, :, :]) * 0.5
    hn18 = jnp.concatenate([hm3[:, 0:16, :], jnp.zeros((16, 2, 64), F32)], 1)
    HN[O0:O0 + M0, :] = hn18.reshape(M0, 64)

    acc = None
    for di in range(3):
        for dj in range(3):
            st = OH + SI * (di - 1) + (dj - 1)
            d = jnp.dot(H2[st:st + MI, :], wwx[di * 3 + dj],
                        preferred_element_type=F32)
            acc = d if acc is None else acc + d
    i0v = jnp.where(m66, _aff_clip(acc, af_wx), zf)
    i4 = i0v.reshape(32, 33, 2, 64)
    for b in range(2):
        i5 = i4[:, :, b, :].reshape(16, 2, 33, 64)
        for a in range(2):
            I0[a * 2 + b, :, :] = i5[:, a, 0:18, :].reshape(M0, 64)

    acc = None
    for di in range(3):
        for dj in range(3):
            st = O0 + S0 * (di - 1) + (dj - 1)
            d = jnp.dot(HN[st:st + M0, :], wwxn[di * 3 + dj],
                        preferred_element_type=F32)
            acc = d if acc is None else acc + d
    i1v = jnp.where(m18, _aff_clip(acc, af_wxn), zf)
    i6 = i1v.reshape(16, 9, 2, 64)
    for b in range(2):
        i7 = i6[:, :, b, :].reshape(8, 2, 9, 64)
        for a in range(2):
            v = jnp.concatenate([i7[:, a, :, :], jnp.zeros((8, 1, 64), F32)], 1)
            I1[a * 2 + b, :, :] = v.reshape(M1, 64)

    def half_step0():
        for a in range(2):
            for b in range(2):
                acc = _conv1_phase(A0, ws1, a, b, S0, O0, M0)
                S1b[a * 2 + b, O0:O0 + M0, :] = jnp.where(
                    m18s1, _aff_clip(acc, af_s1), zf)
        acc = _conv2_full(S1b, ws2, S0, O0, M0)
        S2b[O0:O0 + M0, :] = jnp.where(m18s2, _aff_clip(acc, af_s2), zf)

    def half_step1():
        for a in range(2):
            for b in range(2):
                acc = _conv1_phase(A1, ws1n, a, b, S1, O1, M1)
                S1n[a * 2 + b, O1:O1 + M1, :] = jnp.where(
                    m10s1, _aff_clip(acc, af_s1n), zf)
        acc = _conv2_full(S1n, ws2n, S1, O1, M1)
        S2n[O1:O1 + M1, :] = jnp.where(m10s2, _aff_clip(acc, af_s2n), zf)

    def step(_, carry):
        half_step0()
        half_step1()
        for e in range(2):
            for f in range(2):
                t0 = _convt_phase(S2b, wtr, e, f, S0, O0, M0)
                A0[e * 2 + f, O0:O0 + M0, :] = jnp.where(
                    m18, jnp.clip(t0 + I0[e * 2 + f, :, :], 0.0, VTH_), zf)
                t1 = _convt_phase(S2n, wtrn, e, f, S1, O1, M1)
                A1[e * 2 + f, O1:O1 + M1, :] = jnp.where(
                    m10, jnp.clip(t1 + I1[e * 2 + f, :, :], 0.0, VTH_), zf)
        return carry

    lax.fori_loop(0, T_, step, 0)

    half_step0()
    half_step1()

    s2v = S2b[...].reshape(168, 2, 256)
    e2, o2 = s2v[:, 0, :], s2v[:, 1, :]
    acc = None
    for di in range(3):
        for dj in range(3):
            off = S0 * di + dj + (O0 - S0 - 1)
            src = o2 if (off & 1) else e2
            b0 = off // 2
            d = jnp.dot(src[b0:b0 + 144, :], wds[di * 3 + dj],
                        preferred_element_type=F32)
            acc = d if acc is None else acc + d
    dsv = _aff_clip(acc, af_ds)
    dsc = dsv.reshape(8, 18, 256)[:, 0:8, :].reshape(64, 256)
    z1c = S2n[O1:O1 + M1, :].reshape(8, 10, 256)[:, 0:8, :].reshape(64, 256)
    z = dsc + z1c
    zq = _aff_clip(jnp.dot(z, winc[...], preferred_element_type=F32), af_inc)
    out[0, :, :] = zq


def _cls_kernel(z, w, bias, out):
    out[...] = jnp.dot(z[...], w[...], preferred_element_type=F32) + bias[...]


def _tap9(w):
    return jnp.transpose(w, (2, 3, 1, 0)).reshape(9, w.shape[1], w.shape[0])


def _aff(p, bkey, bnkey):
    g = p[bnkey]['g']
    bb = p[bnkey]['b']
    bias = p[bkey] if bkey is not None else jnp.zeros_like(bb)
    return jnp.stack([g, bias * g + bb])


def kernel(x, params):
    p = params
    B = x.shape[0]

    xp = jnp.pad(x, ((0, 0), (0, 0), (1, 1), (1, 1)))
    pats = jnp.stack([xp[:, :, di:di + 64, dj:dj + 64]
                      for di in range(3) for dj in range(3)], axis=1)
    pats = pats.transpose(0, 3, 4, 1, 2).reshape(B, 64, 64, 27)
    pats = jnp.pad(pats, ((0, 0), (1, 1), (1, 1), (0, 5)))
    x9 = pats.reshape(B, RH, 32)

    w27 = jnp.transpose(p['di_w1'], (2, 3, 1, 0)).reshape(27, 64)
    w27 = jnp.concatenate([w27, jnp.zeros((5, 64), F32)], 0)

    def tr9(w):
        wf = jnp.flip(w, (2, 3)).transpose(1, 0, 2, 3)
        return _tap9(wf)

    weights = [
        w27, _aff(p, 'di_b1', 'di_bn1'),
        _tap9(p['di_w2']), _aff(p, 'di_b2', 'di_bn2'),
        _tap9(p['wx_w']), _aff(p, 'wx_b', 'wx_bn'),
        _tap9(p['wxn_w']), _aff(p, 'wxn_b', 'wxn_bn'),
        _tap9(p['s1_w']), _aff(p, 's1_b', 's1_bn'),
        _tap9(p['s2_w']), _aff(p, 's2_b', 's2_bn'),
        _tap9(p['s1n_w']), _aff(p, 's1n_b', 's1n_bn'),
        _tap9(p['s2n_w']), _aff(p, 's2n_b', 's2n_bn'),
        tr9(p['tr_w']), tr9(p['trn_w']),
        _tap9(p['ds_w']), _aff(p, 'ds_b', 'ds_bn'),
        p['inc_w'][:, :, 0, 0].T, _aff(p, None, 'inc_bn'),
    ]

    wspecs = [pl.BlockSpec(w.shape, lambda i, nd=w.ndim: (0,) * nd)
              for w in weights]
    scratch = [
        pltpu.VMEM((REO, 64), F32),
        pltpu.VMEM((REO, 64), F32),
        pltpu.VMEM((RH, 64), F32),
        pltpu.VMEM((R0, 64), F32),
        pltpu.VMEM((4, R0, 64), F32),
        pltpu.VMEM((4, R0, 128), F32),
        pltpu.VMEM((R0, 256), F32),
        pltpu.VMEM((4, M0, 64), F32),
        pltpu.VMEM((4, R1, 64), F32),
        pltpu.VMEM((4, R1, 128), F32),
        pltpu.VMEM((R1, 256), F32),
        pltpu.VMEM((4, M1, 64), F32),
    ]
    zmap = pl.pallas_call(
        _main_kernel,
        grid=(B,),
        in_specs=[pl.BlockSpec((1, RH, 32), lambda i: (i, 0, 0))] + wspecs,
        out_specs=pl.BlockSpec((1, 64, 256), lambda i: (i, 0, 0)),
        out_shape=jax.ShapeDtypeStruct((B, 64, 256), F32),
        scratch_shapes=scratch,
        compiler_params=pltpu.CompilerParams(
            dimension_semantics=("parallel",),
            vmem_limit_bytes=100 * 1024 * 1024,
        ),
    )(x9, *weights)

    zflat = zmap.reshape(B, 64 * 256)
    wc = p['cls_w'].reshape(100, 256, 64).transpose(2, 1, 0).reshape(16384, 100)
    logits = pl.pallas_call(
        _cls_kernel,
        grid=(2,),
        in_specs=[
            pl.BlockSpec((B // 2, 16384), lambda i: (i, 0)),
            pl.BlockSpec((16384, 100), lambda i: (0, 0)),
            pl.BlockSpec((1, 100), lambda i: (0, 0)),
        ],
        out_specs=pl.BlockSpec((B // 2, 100), lambda i: (i, 0)),
        out_shape=jax.ShapeDtypeStruct((B, 100), F32),
        compiler_params=pltpu.CompilerParams(
            dimension_semantics=("parallel",),
        ),
    )(zflat, wc, p['cls_b'][None, :])
    return logits


# speedup vs baseline: 1.0070x; 1.0070x over previous
import jax
import jax.numpy as jnp
from jax import lax
from jax.experimental import pallas as pl
from jax.experimental.pallas import tpu as pltpu

VTH_ = 1.0
T_ = 8

F32 = jnp.float32

S0, O0, M0, R0 = 18, 24, 288, 336
S1, O1, M1, R1 = 10, 16, 80, 112
SI, MI = 66, 2112
RH = 4356
OH = 72
REO = 2184


def _rowmask(m, s, v, c):
    rf = lax.broadcasted_iota(jnp.int32, (m, c), 0).astype(F32)
    pj = rf - jnp.floor((rf + 0.5) * (1.0 / s)) * s
    return pj < v


def _aff_clip(acc, af_ref):
    return jnp.clip(acc * af_ref[0:1, :] + af_ref[1:2, :], 0.0, VTH_)


def _conv1_phase(src, w_ref, a, b, s, o, m):
    acc = None
    for di in range(3):
        qa = a + di - 1
        pa, du = qa & 1, (qa - (qa & 1)) // 2
        for dj in range(3):
            qb = b + dj - 1
            pb, dv = qb & 1, (qb - (qb & 1)) // 2
            st = o + s * du + dv
            d = jnp.dot(src[pa * 2 + pb, st:st + m, :], w_ref[di * 3 + dj],
                        preferred_element_type=F32)
            acc = d if acc is None else acc + d
    return acc


def _conv2_full(src, w_ref, s, o, m):
    acc = None
    for di in range(3):
        q = di - 1
        pa, du = q & 1, (q - (q & 1)) // 2
        for dj in range(3):
            q2 = dj - 1
            pb, dv = q2 & 1, (q2 - (q2 & 1)) // 2
            st = o + s * du + dv
            d = jnp.dot(src[pa * 2 + pb, st:st + m, :], w_ref[di * 3 + dj],
                        preferred_element_type=F32)
            acc = d if acc is None else acc + d
    return acc


def _convt_phase(s2, w_ref, e, f, s, o, m):
    acc = None
    for di in range(3):
        if (di & 1) == (e & 1):
            continue
        du = (e + di - 1) // 2
        for dj in range(3):
            if (dj & 1) == (f & 1):
                continue
            dv = (f + dj - 1) // 2
            st = o + s * du + dv
            d = jnp.dot(s2[st:st + m, :], w_ref[di * 3 + dj],
                        preferred_element_type=F32)
            acc = d if acc is None else acc + d
    return acc


def _main_kernel(x9, w27, af_di1, wdi2, af_di2, wwx, af_wx, wwxn, af_wxn,
                 ws1, af_s1, ws2, af_s2, ws1n, af_s1n, ws2n, af_s2n,
                 wtr, wtrn, wds, af_ds, winc, af_inc, out,
                 E, Ob, H2, HN, A0, S1b, S2b, I0, A1, S1n, S2n, I1):
    m18 = _rowmask(M0, S0, 16, 64)
    m18s2 = _rowmask(M0, S0, 16, 256)
    m18s1 = _rowmask(M0, S0, 16, 128)
    m10 = _rowmask(M1, S1, 8, 64)
    m10s1 = _rowmask(M1, S1, 8, 128)
    m10s2 = _rowmask(M1, S1, 8, 256)
    m66 = _rowmask(MI, SI, 32, 64)
    zf = jnp.float32(0.0)

    for r4 in (A0, S1b, A1, S1n):
        r4[...] = jnp.zeros_like(r4)
    S2b[...] = jnp.zeros_like(S2b)
    S2n[...] = jnp.zeros_like(S2n)
    HN[...] = jnp.zeros_like(HN)
    H2[0:OH, :] = jnp.zeros((OH, 64), F32)
    H2[OH + MI:RH, :] = jnp.zeros((RH - OH - MI, 64), F32)
    E[2178:REO, :] = jnp.zeros((REO - 2178, 64), F32)
    Ob[2178:REO, :] = jnp.zeros((REO - 2178, 64), F32)

    rf = lax.broadcasted_iota(jnp.int32, (RH, 64), 0).astype(F32)
    pj = rf - jnp.floor((rf + 0.5) * (1.0 / SI)) * SI
    mint = (rf >= SI) & (rf < RH - SI) & (pj >= 1) & (pj < 65)
    h1 = jnp.dot(x9[0], w27[...], preferred_element_type=F32)
    h1 = jnp.where(mint, _aff_clip(h1, af_di1), zf)
    v3 = h1.reshape(2178, 2, 64)
    E[0:2178, :] = v3[:, 0, :]
    Ob[0:2178, :] = v3[:, 1, :]

    acc = None
    for di in range(3):
        for dj in range(3):
            off = di * SI + dj
            src = Ob if (off & 1) else E
            b0 = off // 2
            d = jnp.dot(src[b0:b0 + MI, :], wdi2[di * 3 + dj],
                        preferred_element_type=F32)
            acc = d if acc is None else acc + d
    hv = jnp.where(m66, _aff_clip(acc, af_di2), zf)
    H2[OH:OH + MI, :] = hv

    h4 = hv.reshape(32, 33, 2, 64)
    hm = (h4[:, :, 0, :] + h4[:, :, 1, :]) * 0.5
    hm2 = hm.reshape(16, 2, 33, 64)
    hm3 = (hm2[:, 0, :, :] + hm2[:, 1, :, :]) * 0.5
    hn18 = jnp.concatenate([hm3[:, 0:16, :], jnp.zeros((16, 2, 64), F32)], 1)
    HN[O0:O0 + M0, :] = hn18.reshape(M0, 64)

    acc = None
    for di in range(3):
        for dj in range(3):
            st = OH + SI * (di - 1) + (dj - 1)
            d = jnp.dot(H2[st:st + MI, :], wwx[di * 3 + dj],
                        preferred_element_type=F32)
            acc = d if acc is None else acc + d
    i0v = jnp.where(m66, _aff_clip(acc, af_wx), zf)
    i4 = i0v.reshape(32, 33, 2, 64)
    for b in range(2):
        i5 = i4[:, :, b, :].reshape(16, 2, 33, 64)
        for a in range(2):
            I0[a * 2 + b, :, :] = i5[:, a, 0:18, :].reshape(M0, 64)

    acc = None
    for di in range(3):
        for dj in range(3):
            st = O0 + S0 * (di - 1) + (dj - 1)
            d = jnp.dot(HN[st:st + M0, :], wwxn[di * 3 + dj],
                        preferred_element_type=F32)
            acc = d if acc is None else acc + d
    i1v = jnp.where(m18, _aff_clip(acc, af_wxn), zf)
    i6 = i1v.reshape(16, 9, 2, 64)
    for b in range(2):
        i7 = i6[:, :, b, :].reshape(8, 2, 9, 64)
        for a in range(2):
            v = jnp.concatenate([i7[:, a, :, :], jnp.zeros((8, 1, 64), F32)], 1)
            I1[a * 2 + b, :, :] = v.reshape(M1, 64)

    def half_step0():
        for a in range(2):
            for b in range(2):
                acc = _conv1_phase(A0, ws1, a, b, S0, O0, M0)
                S1b[a * 2 + b, O0:O0 + M0, :] = jnp.where(
                    m18s1, _aff_clip(acc, af_s1), zf)
        acc = _conv2_full(S1b, ws2, S0, O0, M0)
        S2b[O0:O0 + M0, :] = jnp.where(m18s2, _aff_clip(acc, af_s2), zf)

    def half_step1():
        for a in range(2):
            for b in range(2):
                acc = _conv1_phase(A1, ws1n, a, b, S1, O1, M1)
                S1n[a * 2 + b, O1:O1 + M1, :] = jnp.where(
                    m10s1, _aff_clip(acc, af_s1n), zf)
        acc = _conv2_full(S1n, ws2n, S1, O1, M1)
        S2n[O1:O1 + M1, :] = jnp.where(m10s2, _aff_clip(acc, af_s2n), zf)

    def step(_, carry):
        half_step0()
        half_step1()
        for e in range(2):
            for f in range(2):
                t0 = _convt_phase(S2b, wtr, e, f, S0, O0, M0)
                A0[e * 2 + f, O0:O0 + M0, :] = jnp.where(
                    m18, jnp.clip(t0 + I0[e * 2 + f, :, :], 0.0, VTH_), zf)
                t1 = _convt_phase(S2n, wtrn, e, f, S1, O1, M1)
                A1[e * 2 + f, O1:O1 + M1, :] = jnp.where(
                    m10, jnp.clip(t1 + I1[e * 2 + f, :, :], 0.0, VTH_), zf)
        return carry

    lax.fori_loop(0, T_, step, 0)

    half_step0()
    half_step1()

    s2v = S2b[...].reshape(168, 2, 256)
    e2, o2 = s2v[:, 0, :], s2v[:, 1, :]
    acc = None
    for di in range(3):
        for dj in range(3):
            off = S0 * di + dj + (O0 - S0 - 1)
            src = o2 if (off & 1) else e2
            b0 = off // 2
            d = jnp.dot(src[b0:b0 + 144, :], wds[di * 3 + dj],
                        preferred_element_type=F32)
            acc = d if acc is None else acc + d
    dsv = _aff_clip(acc, af_ds)
    dsc = dsv.reshape(8, 18, 256)[:, 0:8, :].reshape(64, 256)
    z1c = S2n[O1:O1 + M1, :].reshape(8, 10, 256)[:, 0:8, :].reshape(64, 256)
    z = dsc + z1c
    zq = _aff_clip(jnp.dot(z, winc[...], preferred_element_type=F32), af_inc)
    out[0, :, :] = zq


def _cls_kernel(z, w, bias, out):
    out[...] = jnp.dot(z[...], w[...], preferred_element_type=F32) + bias[...]


def _tap9(w):
    return jnp.transpose(w, (2, 3, 1, 0)).reshape(9, w.shape[1], w.shape[0])


def _aff(p, bkey, bnkey):
    g = p[bnkey]['g']
    bb = p[bnkey]['b']
    bias = p[bkey] if bkey is not None else jnp.zeros_like(bb)
    return jnp.stack([g, bias * g + bb])


def kernel(x, params):
    p = params
    B = x.shape[0]

    xp = jnp.pad(x, ((0, 0), (0, 0), (1, 1), (1, 1)))
    pats = jnp.stack([xp[:, :, di:di + 64, dj:dj + 64]
                      for di in range(3) for dj in range(3)], axis=1)
    pats = pats.transpose(0, 3, 4, 1, 2).reshape(B, 64, 64, 27)
    pats = jnp.pad(pats, ((0, 0), (1, 1), (1, 1), (0, 5)))
    x9 = pats.reshape(B, RH, 32)

    w27 = jnp.transpose(p['di_w1'], (2, 3, 1, 0)).reshape(27, 64)
    w27 = jnp.concatenate([w27, jnp.zeros((5, 64), F32)], 0)

    def tr9(w):
        wf = jnp.flip(w, (2, 3)).transpose(1, 0, 2, 3)
        return _tap9(wf)

    weights = [
        w27, _aff(p, 'di_b1', 'di_bn1'),
        _tap9(p['di_w2']), _aff(p, 'di_b2', 'di_bn2'),
        _tap9(p['wx_w']), _aff(p, 'wx_b', 'wx_bn'),
        _tap9(p['wxn_w']), _aff(p, 'wxn_b', 'wxn_bn'),
        _tap9(p['s1_w']), _aff(p, 's1_b', 's1_bn'),
        _tap9(p['s2_w']), _aff(p, 's2_b', 's2_bn'),
        _tap9(p['s1n_w']), _aff(p, 's1n_b', 's1n_bn'),
        _tap9(p['s2n_w']), _aff(p, 's2n_b', 's2n_bn'),
        tr9(p['tr_w']), tr9(p['trn_w']),
        _tap9(p['ds_w']), _aff(p, 'ds_b', 'ds_bn'),
        p['inc_w'][:, :, 0, 0].T, _aff(p, None, 'inc_bn'),
    ]

    wspecs = [pl.BlockSpec(w.shape, lambda i, nd=w.ndim: (0,) * nd)
              for w in weights]
    scratch = [
        pltpu.VMEM((REO, 64), F32),
        pltpu.VMEM((REO, 64), F32),
        pltpu.VMEM((RH, 64), F32),
        pltpu.VMEM((R0, 64), F32),
        pltpu.VMEM((4, R0, 64), F32),
        pltpu.VMEM((4, R0, 128), F32),
        pltpu.VMEM((R0, 256), F32),
        pltpu.VMEM((4, M0, 64), F32),
        pltpu.VMEM((4, R1, 64), F32),
        pltpu.VMEM((4, R1, 128), F32),
        pltpu.VMEM((R1, 256), F32),
        pltpu.VMEM((4, M1, 64), F32),
    ]
    zmap = pl.pallas_call(
        _main_kernel,
        grid=(B,),
        in_specs=[pl.BlockSpec((1, RH, 32), lambda i: (i, 0, 0))] + wspecs,
        out_specs=pl.BlockSpec((1, 64, 256), lambda i: (i, 0, 0)),
        out_shape=jax.ShapeDtypeStruct((B, 64, 256), F32),
        scratch_shapes=scratch,
        compiler_params=pltpu.CompilerParams(
            dimension_semantics=("parallel",),
            vmem_limit_bytes=100 * 1024 * 1024,
        ),
    )(x9, *weights)

    zflat = zmap.reshape(B, 64 * 256)
    wc = p['cls_w'].reshape(100, 256, 64).transpose(2, 1, 0).reshape(16384, 100)
    logits = pl.pallas_call(
        _cls_kernel,
        grid=(2,),
        in_specs=[
            pl.BlockSpec((B // 2, 16384), lambda i: (i, 0)),
            pl.BlockSpec((16384, 100), lambda i: (0, 0)),
            pl.BlockSpec((1, 100), lambda i: (0, 0)),
        ],
        out_specs=pl.BlockSpec((B // 2, 100), lambda i: (i, 0)),
        out_shape=jax.ShapeDtypeStruct((B, 100), F32),
        compiler_params=pltpu.CompilerParams(
            dimension_semantics=("parallel",),
        ),
    )(zflat, wc, p['cls_b'][None, :])
    return logits
